# Initial kernel scaffold; baseline (speedup 1.0000x reference)
import jax
import jax.numpy as jnp
from jax.experimental import pallas as pl


def kernel(x, edge_index, edge_attr, batch, W_enc, b_enc, W_conv, b_conv, W_edge, W_pred, b_pred):
    raise NotImplementedError("write your pallas kernel here")


# speedup vs baseline: 4.2037x; 4.2037x over previous
import functools

import jax
import jax.numpy as jnp
from jax import lax
from jax.experimental import pallas as pl
from jax.experimental.pallas import tpu as pltpu
from jax.experimental.pallas import tpu_sc as plsc

N_NODES = 10000
N_EDGES = 320000
D = 128
D_EDGE = 4
G_GRAPHS = 128
NUM_TASKS = 1

NC = 2
NS = 16
L = 16

EB = 128
NBLK = 2560
E_PAD = NBLK * EB
BPT = NBLK // NS
NPAD = 10112
RPS = NPAD // NS
DUMMY_ROW = N_NODES + 8
W16 = 16

_mesh = plsc.VectorSubcoreMesh(core_axis_name="c", subcore_axis_name="s")


def _zero_rows(ref, nrows, width):
    @pl.loop(0, nrows)
    def _(r):
        for j in range(width // L):
            ref[r, pl.ds(j * L, L)] = jnp.zeros((L,), jnp.float32)


def _zero_acc_slice(zsrc, acc, base):
    for k in range(RPS // EB):
        pltpu.sync_copy(zsrc, acc.at[pl.ds(base + k * EB, EB)])
    rem = RPS % EB
    if rem:
        pltpu.sync_copy(zsrc.at[pl.ds(0, rem)],
                        acc.at[pl.ds(base + (RPS // EB) * EB, rem)])


CHUNK = 16
BPT2 = NBLK // (NC * NS)


@functools.partial(
    pl.kernel,
    out_type=jax.ShapeDtypeStruct((NC, NPAD, D), jnp.float32),
    mesh=_mesh,
    scratch_types=[
        pltpu.VMEM((CHUNK, EB), jnp.int32),
        pltpu.VMEM((EB, D), jnp.float32),
        pltpu.VMEM_SHARED((NPAD, D), jnp.float32),
    ],
)
def _deg_kernel(dst_hbm, deg_hbm, dstv, ones, acc):
    c = lax.axis_index("c")
    s = lax.axis_index("s")
    w = s * NC + c
    _zero_rows(ones, EB, D)
    _zero_acc_slice(ones, acc, s * RPS)

    @pl.loop(0, EB)
    def _(r):
        for j in range(D // L):
            ones[r, pl.ds(j * L, L)] = jnp.full((L,), 1.0, jnp.float32)

    plsc.subcore_barrier()

    @pl.loop(0, BPT2 // CHUNK)
    def _(g):
        pltpu.sync_copy(dst_hbm.at[pl.ds(w * BPT2 + g * CHUNK, CHUNK)], dstv)

        @pl.loop(0, CHUNK)
        def _(b):
            pltpu.sync_copy(ones, acc.at[dstv.at[b]], add=True)

    plsc.subcore_barrier()

    @pl.when(c == 0)
    def _():
        pltpu.sync_copy(acc.at[pl.ds(s * RPS, RPS)],
                        deg_hbm.at[0, pl.ds(s * RPS, RPS)])

    @pl.when(c == 1)
    def _():
        pltpu.sync_copy(acc.at[pl.ds(s * RPS, RPS)],
                        deg_hbm.at[1, pl.ds(s * RPS, RPS)])


@functools.partial(
    pl.kernel,
    out_type=jax.ShapeDtypeStruct((NC, NPAD, D), jnp.float32),
    mesh=_mesh,
    scratch_types=[
        pltpu.VMEM((CHUNK, EB), jnp.int32),
        pltpu.VMEM((CHUNK, EB), jnp.int32),
        pltpu.VMEM((EB, D), jnp.float32),
        pltpu.VMEM((W16, EB), jnp.float32),
        pltpu.VMEM((EB, D), jnp.float32),
        pltpu.SemaphoreType.DMA,
        pltpu.VMEM_SHARED((NPAD, D), jnp.float32),
    ],
)
def _edge_agg_kernel(norm_hbm, src_hbm, dst_hbm, ea_hbm, a_hbm,
                     srcv, dstv, nr, eav, t2, sem, acc):
    c = lax.axis_index("c")
    s = lax.axis_index("s")
    w = s * NC + c
    _zero_rows(t2, EB, D)
    _zero_acc_slice(t2, acc, s * RPS)
    plsc.subcore_barrier()

    @pl.loop(0, BPT2 // CHUNK)
    def _(g):
        base = w * BPT2 + g * CHUNK
        pltpu.sync_copy(src_hbm.at[pl.ds(base, CHUNK)], srcv)
        pltpu.sync_copy(dst_hbm.at[pl.ds(base, CHUNK)], dstv)

        @pl.loop(0, CHUNK)
        def _(b):
            pltpu.sync_copy(ea_hbm.at[pl.ds((base + b) * W16, W16)], eav)
            pltpu.async_copy(norm_hbm.at[srcv.at[b]], nr, sem).wait()

            @pl.loop(0, W16)
            def _(r):
                for q in range(EB // W16):
                    e = r * (EB // W16) + q
                    t2[e, pl.ds(0, W16)] = (
                        nr[e, pl.ds(0, W16)] * eav[r, pl.ds(q * W16, W16)])

            pltpu.sync_copy(t2, acc.at[dstv.at[b]], add=True)

    plsc.subcore_barrier()

    @pl.when(c == 0)
    def _():
        pltpu.sync_copy(acc.at[pl.ds(s * RPS, RPS)],
                        a_hbm.at[0, pl.ds(s * RPS, RPS)])

    @pl.when(c == 1)
    def _():
        pltpu.sync_copy(acc.at[pl.ds(s * RPS, RPS)],
                        a_hbm.at[1, pl.ds(s * RPS, RPS)])


@functools.partial(
    pl.kernel,
    out_type=jax.ShapeDtypeStruct((NC, NPAD, D), jnp.float32),
    mesh=_mesh,
    scratch_types=[
        pltpu.VMEM((CHUNK, EB), jnp.int32),
        pltpu.VMEM((CHUNK, EB), jnp.int32),
        pltpu.VMEM((EB, D), jnp.float32),
        pltpu.VMEM((EB, D), jnp.float32),
        pltpu.SemaphoreType.DMA,
        pltpu.SemaphoreType.DMA,
        pltpu.VMEM_SHARED((NPAD, D), jnp.float32),
    ],
)
def _spmm_kernel(u0_hbm, u1_hbm, src_hbm, dst_hbm, s_hbm,
                 srcv, dstv, rows0, rows1, sem0, sem1, acc):
    c = lax.axis_index("c")
    s = lax.axis_index("s")
    _zero_rows(rows0, EB, D)
    _zero_acc_slice(rows0, acc, s * RPS)
    plsc.subcore_barrier()

    def run(u_hbm):
        @pl.loop(0, BPT // CHUNK)
        def _(g):
            base = s * BPT + g * CHUNK
            pltpu.sync_copy(src_hbm.at[pl.ds(base, CHUNK)], srcv)
            pltpu.sync_copy(dst_hbm.at[pl.ds(base, CHUNK)], dstv)

            @pl.loop(0, CHUNK, step=2)
            def _(b):
                cp0 = pltpu.async_copy(u_hbm.at[srcv.at[b]], rows0, sem0)
                cp1 = pltpu.async_copy(u_hbm.at[srcv.at[b + 1]], rows1, sem1)
                cp0.wait()
                pltpu.sync_copy(rows0, acc.at[dstv.at[b]], add=True)
                cp1.wait()
                pltpu.sync_copy(rows1, acc.at[dstv.at[b + 1]], add=True)

    @pl.when(c == 0)
    def _():
        run(u0_hbm)

    @pl.when(c == 1)
    def _():
        run(u1_hbm)

    plsc.subcore_barrier()

    @pl.when(c == 0)
    def _():
        pltpu.sync_copy(acc.at[pl.ds(s * RPS, RPS)],
                        s_hbm.at[0, pl.ds(s * RPS, RPS)])

    @pl.when(c == 1)
    def _():
        pltpu.sync_copy(acc.at[pl.ds(s * RPS, RPS)],
                        s_hbm.at[1, pl.ds(s * RPS, RPS)])


_RB = 1000


def _prep_body(x_ref, we_ref, be_ref, deg_ref, wc0_ref, wc1_ref,
               norm_ref, u0_ref, u1_ref):
    deg = deg_ref[0][:, 0:1] + deg_ref[1][:, 0:1]
    norm = lax.rsqrt(jnp.maximum(deg, 1.0))
    norm_ref[...] = jnp.broadcast_to(norm, (_RB, D))
    h0 = jnp.dot(x_ref[...], we_ref[...],
                 preferred_element_type=jnp.float32) + be_ref[...]
    hn = h0 * norm
    u0_ref[...] = jnp.dot(hn, wc0_ref[...], preferred_element_type=jnp.float32)
    u1_ref[...] = jnp.dot(hn, wc1_ref[...], preferred_element_type=jnp.float32)


def _tc_prep(x, w_enc, b_enc, deg16, wc0, wc1):
    grid = (N_NODES // _RB,)
    call = pl.pallas_call(
        _prep_body,
        grid=grid,
        in_specs=[
            pl.BlockSpec((_RB, D), lambda i: (i, 0)),
            pl.BlockSpec((D, D), lambda i: (0, 0)),
            pl.BlockSpec((1, D), lambda i: (0, 0)),
            pl.BlockSpec((2, _RB, D), lambda i: (0, i, 0)),
            pl.BlockSpec((D, D), lambda i: (0, 0)),
            pl.BlockSpec((D, D), lambda i: (0, 0)),
        ],
        out_specs=[
            pl.BlockSpec((_RB, D), lambda i: (i, 0)),
            pl.BlockSpec((_RB, D), lambda i: (i, 0)),
            pl.BlockSpec((_RB, D), lambda i: (i, 0)),
        ],
        out_shape=[
            jax.ShapeDtypeStruct((N_NODES, D), jnp.float32),
            jax.ShapeDtypeStruct((N_NODES, D), jnp.float32),
            jax.ShapeDtypeStruct((N_NODES, D), jnp.float32),
        ],
    )
    return call(x, w_enc, b_enc, deg16, wc0, wc1)


def _mid_body(s_ref, a_ref, norm_ref, wep_ref, bc_ref, wc_ref,
              u0_ref, u1_ref):
    nc = norm_ref[...][:, 0:1]
    a16 = a_ref[0] + a_ref[1]
    outs = (u0_ref, u1_ref)
    for i in range(2):
        agg = nc * (s_ref[i] + jnp.dot(a16, wep_ref[i],
                                       preferred_element_type=jnp.float32))
        h = jax.nn.relu(agg + bc_ref[i])
        outs[i][...] = jnp.dot(h * nc, wc_ref[i],
                               preferred_element_type=jnp.float32)


def _tc_mid(s2, a16, norm16, wep, bc, wc):
    grid = (N_NODES // _RB,)
    return pl.pallas_call(
        _mid_body,
        grid=grid,
        in_specs=[
            pl.BlockSpec((2, _RB, D), lambda i: (0, i, 0)),
            pl.BlockSpec((2, _RB, D), lambda i: (0, i, 0)),
            pl.BlockSpec((_RB, D), lambda i: (i, 0)),
            pl.BlockSpec((2, D, D), lambda i: (0, 0, 0)),
            pl.BlockSpec((2, 1, D), lambda i: (0, 0, 0)),
            pl.BlockSpec((2, D, D), lambda i: (0, 0, 0)),
        ],
        out_specs=[
            pl.BlockSpec((_RB, D), lambda i: (i, 0)),
            pl.BlockSpec((_RB, D), lambda i: (i, 0)),
        ],
        out_shape=[
            jax.ShapeDtypeStruct((N_NODES, D), jnp.float32),
            jax.ShapeDtypeStruct((N_NODES, D), jnp.float32),
        ],
    )(s2, a16, norm16, wep, bc, wc)


def _final_body(s_ref, a_ref, norm_ref, wep_ref, bc_ref, batch_ref,
                wpred_ref, bpred_ref, out_ref):
    nc = norm_ref[...][:, 0:1]
    a16 = a_ref[0] + a_ref[1]
    hsum = jnp.zeros((N_NODES, D), jnp.float32)
    for i in range(2):
        agg = nc * (s_ref[i] + jnp.dot(a16, wep_ref[i],
                                       preferred_element_type=jnp.float32))
        hsum = hsum + jax.nn.relu(agg + bc_ref[i])
    gids = lax.broadcasted_iota(jnp.int32, (G_GRAPHS, N_NODES), 0)
    m = (gids == batch_ref[...]).astype(jnp.float32)
    sums = jnp.dot(m, hsum, preferred_element_type=jnp.float32)
    counts = jnp.sum(m, axis=1, keepdims=True)
    readout = sums / jnp.maximum(counts, 1.0)
    out_ref[...] = jnp.dot(readout, wpred_ref[...],
                           preferred_element_type=jnp.float32) + bpred_ref[...]


def _tc_final(s2, a16, norm16, wep, bc, batch2d, wpred_p, bpred_p):
    return pl.pallas_call(
        _final_body,
        out_shape=jax.ShapeDtypeStruct((G_GRAPHS, D), jnp.float32),
    )(s2, a16, norm16, wep, bc, batch2d, wpred_p, bpred_p)


def kernel(x, edge_index, edge_attr, batch,
           W_enc, b_enc, W_conv, b_conv, W_edge, W_pred, b_pred):
    src = edge_index[0]
    dst = edge_index[1]
    pad = E_PAD - N_EDGES
    src_p = jnp.concatenate(
        [src, jnp.zeros((pad,), jnp.int32)]).reshape(NBLK, EB)
    dst_p = jnp.concatenate(
        [dst, jnp.full((pad,), DUMMY_ROW, jnp.int32)]).reshape(NBLK, EB)
    ea_r = jnp.zeros((E_PAD, W16), jnp.float32).at[:N_EDGES, :D_EDGE].set(
        edge_attr).reshape(NBLK * W16, EB)

    deg2 = _deg_kernel(dst_p)[:, :N_NODES]
    norm128, u0, u1 = _tc_prep(x, W_enc, b_enc.reshape(1, D), deg2,
                               W_conv[0, 0], W_conv[1, 0])

    a2 = _edge_agg_kernel(norm128, src_p, dst_p, ea_r)[:, :N_NODES]

    wep = jnp.zeros((2, 2, D, D), jnp.float32).at[:, :, :D_EDGE, :].set(W_edge)
    bc = b_conv.reshape(2, 2, 1, D)

    s2 = _spmm_kernel(u0, u1, src_p, dst_p)[:, :N_NODES]
    u0n, u1n = _tc_mid(s2, a2, norm128, wep[:, 0], bc[:, 0],
                       jnp.stack([W_conv[0, 1], W_conv[1, 1]]))

    s2b = _spmm_kernel(u0n, u1n, src_p, dst_p)[:, :N_NODES]

    wpred_p = jnp.zeros((D, D), jnp.float32).at[:, :NUM_TASKS].set(W_pred)
    bpred_p = jnp.zeros((1, D), jnp.float32).at[0, :NUM_TASKS].set(b_pred)
    out = _tc_final(s2b, a2, norm128, wep[:, 1], bc[:, 1],
                    batch.reshape(1, N_NODES), wpred_p, bpred_p)
    return out[:, :NUM_TASKS]


# speedup vs baseline: 4.4688x; 1.0631x over previous
import functools

import jax
import jax.numpy as jnp
from jax import lax
from jax.experimental import pallas as pl
from jax.experimental.pallas import tpu as pltpu
from jax.experimental.pallas import tpu_sc as plsc

N_NODES = 10000
N_EDGES = 320000
D = 128
D_EDGE = 4
G_GRAPHS = 128
NUM_TASKS = 1

NC = 2
NS = 16
L = 16

EB = 128
NBLK = 2560
E_PAD = NBLK * EB
BPT = NBLK // NS
NPAD = 10112
RPS = NPAD // NS
DUMMY_ROW = N_NODES + 8
W16 = 16

_mesh = plsc.VectorSubcoreMesh(core_axis_name="c", subcore_axis_name="s")


def _zero_rows(ref, nrows, width):
    @pl.loop(0, nrows)
    def _(r):
        for j in range(width // L):
            ref[r, pl.ds(j * L, L)] = jnp.zeros((L,), jnp.float32)


def _zero_acc_slice(zsrc, acc, base):
    for k in range(RPS // EB):
        pltpu.sync_copy(zsrc, acc.at[pl.ds(base + k * EB, EB)])
    rem = RPS % EB
    if rem:
        pltpu.sync_copy(zsrc.at[pl.ds(0, rem)],
                        acc.at[pl.ds(base + (RPS // EB) * EB, rem)])


CHUNK = 16
BPT2 = NBLK // (NC * NS)


@functools.partial(
    pl.kernel,
    out_type=jax.ShapeDtypeStruct((NC, NPAD, D), jnp.float32),
    mesh=_mesh,
    scratch_types=[
        pltpu.VMEM((CHUNK, EB), jnp.int32),
        pltpu.VMEM((EB, D), jnp.float32),
        pltpu.VMEM_SHARED((NPAD, D), jnp.float32),
    ],
)
def _deg_kernel(dst_hbm, deg_hbm, dstv, ones, acc):
    c = lax.axis_index("c")
    s = lax.axis_index("s")
    w = s * NC + c
    _zero_rows(ones, EB, D)
    _zero_acc_slice(ones, acc, s * RPS)

    @pl.loop(0, EB)
    def _(r):
        for j in range(D // L):
            ones[r, pl.ds(j * L, L)] = jnp.full((L,), 1.0, jnp.float32)

    plsc.subcore_barrier()

    @pl.loop(0, BPT2 // CHUNK)
    def _(g):
        pltpu.sync_copy(dst_hbm.at[pl.ds(w * BPT2 + g * CHUNK, CHUNK)], dstv)

        @pl.loop(0, CHUNK)
        def _(b):
            pltpu.sync_copy(ones, acc.at[dstv.at[b]], add=True)

    plsc.subcore_barrier()

    @pl.when(c == 0)
    def _():
        pltpu.sync_copy(acc.at[pl.ds(s * RPS, RPS)],
                        deg_hbm.at[0, pl.ds(s * RPS, RPS)])

    @pl.when(c == 1)
    def _():
        pltpu.sync_copy(acc.at[pl.ds(s * RPS, RPS)],
                        deg_hbm.at[1, pl.ds(s * RPS, RPS)])


@functools.partial(
    pl.kernel,
    out_type=jax.ShapeDtypeStruct((NC, NPAD, D), jnp.float32),
    mesh=_mesh,
    scratch_types=[
        pltpu.VMEM((CHUNK, EB), jnp.int32),
        pltpu.VMEM((CHUNK, EB), jnp.int32),
        pltpu.VMEM((EB, D), jnp.float32),
        pltpu.VMEM((W16, EB), jnp.float32),
        pltpu.VMEM((EB, D), jnp.float32),
        pltpu.SemaphoreType.DMA,
        pltpu.VMEM_SHARED((NPAD, D), jnp.float32),
    ],
)
def _edge_agg_kernel(norm_hbm, src_hbm, dst_hbm, ea_hbm, a_hbm,
                     srcv, dstv, nr, eav, t2, sem, acc):
    c = lax.axis_index("c")
    s = lax.axis_index("s")
    w = s * NC + c
    _zero_rows(t2, EB, D)
    _zero_acc_slice(t2, acc, s * RPS)
    plsc.subcore_barrier()

    @pl.loop(0, BPT2 // CHUNK)
    def _(g):
        base = w * BPT2 + g * CHUNK
        pltpu.sync_copy(src_hbm.at[pl.ds(base, CHUNK)], srcv)
        pltpu.sync_copy(dst_hbm.at[pl.ds(base, CHUNK)], dstv)

        @pl.loop(0, CHUNK)
        def _(b):
            pltpu.sync_copy(ea_hbm.at[pl.ds((base + b) * W16, W16)], eav)
            pltpu.async_copy(norm_hbm.at[srcv.at[b]], nr, sem).wait()

            @pl.loop(0, W16)
            def _(r):
                for q in range(EB // W16):
                    e = r * (EB // W16) + q
                    t2[e, pl.ds(0, W16)] = (
                        nr[e, pl.ds(0, W16)] * eav[r, pl.ds(q * W16, W16)])

            pltpu.sync_copy(t2, acc.at[dstv.at[b]], add=True)

    plsc.subcore_barrier()

    @pl.when(c == 0)
    def _():
        pltpu.sync_copy(acc.at[pl.ds(s * RPS, RPS)],
                        a_hbm.at[0, pl.ds(s * RPS, RPS)])

    @pl.when(c == 1)
    def _():
        pltpu.sync_copy(acc.at[pl.ds(s * RPS, RPS)],
                        a_hbm.at[1, pl.ds(s * RPS, RPS)])


@functools.partial(
    pl.kernel,
    out_type=jax.ShapeDtypeStruct((NC, NPAD, D), jnp.float32),
    mesh=_mesh,
    scratch_types=[
        pltpu.VMEM((CHUNK, EB), jnp.int32),
        pltpu.VMEM((CHUNK, EB), jnp.int32),
        pltpu.VMEM((EB, D), jnp.float32),
        pltpu.VMEM((EB, D), jnp.float32),
        pltpu.SemaphoreType.DMA,
        pltpu.SemaphoreType.DMA,
        pltpu.SemaphoreType.DMA,
        pltpu.SemaphoreType.DMA,
        pltpu.VMEM_SHARED((NPAD, D), jnp.float32),
    ],
)
def _spmm_kernel(u0_hbm, u1_hbm, src_hbm, dst_hbm, s_hbm,
                 srcv, dstv, rows0, rows1, gsem0, gsem1, ssem0, ssem1, acc):
    c = lax.axis_index("c")
    s = lax.axis_index("s")
    _zero_rows(rows0, EB, D)
    _zero_acc_slice(rows0, acc, s * RPS)
    plsc.subcore_barrier()

    def run(u_hbm):
        @pl.loop(0, BPT // CHUNK)
        def _(g):
            base = s * BPT + g * CHUNK
            pltpu.sync_copy(src_hbm.at[pl.ds(base, CHUNK)], srcv)
            pltpu.sync_copy(dst_hbm.at[pl.ds(base, CHUNK)], dstv)
            pltpu.async_copy(u_hbm.at[srcv.at[0]], rows0, gsem0)

            @pl.loop(0, CHUNK, step=2)
            def _(b):
                @pl.when(b > 0)
                def _():
                    pltpu.make_async_copy(
                        rows1, acc.at[dstv.at[b]], ssem1).wait()
                pltpu.async_copy(u_hbm.at[srcv.at[b + 1]], rows1, gsem1)
                pltpu.make_async_copy(
                    u_hbm.at[srcv.at[b]], rows0, gsem0).wait()
                pltpu.async_copy(rows0, acc.at[dstv.at[b]], ssem0, add=True)

                @pl.when(b + 2 < CHUNK)
                def _():
                    pltpu.make_async_copy(
                        rows0, acc.at[dstv.at[b]], ssem0).wait()
                    pltpu.async_copy(u_hbm.at[srcv.at[b + 2]], rows0, gsem0)
                pltpu.make_async_copy(
                    u_hbm.at[srcv.at[b + 1]], rows1, gsem1).wait()
                pltpu.async_copy(rows1, acc.at[dstv.at[b + 1]], ssem1, add=True)

            pltpu.make_async_copy(rows0, acc.at[dstv.at[0]], ssem0).wait()
            pltpu.make_async_copy(rows1, acc.at[dstv.at[1]], ssem1).wait()

    @pl.when(c == 0)
    def _():
        run(u0_hbm)

    @pl.when(c == 1)
    def _():
        run(u1_hbm)

    plsc.subcore_barrier()

    @pl.when(c == 0)
    def _():
        pltpu.sync_copy(acc.at[pl.ds(s * RPS, RPS)],
                        s_hbm.at[0, pl.ds(s * RPS, RPS)])

    @pl.when(c == 1)
    def _():
        pltpu.sync_copy(acc.at[pl.ds(s * RPS, RPS)],
                        s_hbm.at[1, pl.ds(s * RPS, RPS)])


_RB = 1000


def _prep_body(x_ref, we_ref, be_ref, deg_ref, wc0_ref, wc1_ref,
               norm_ref, u0_ref, u1_ref):
    deg = deg_ref[0][:, 0:1] + deg_ref[1][:, 0:1]
    norm = lax.rsqrt(jnp.maximum(deg, 1.0))
    norm_ref[...] = jnp.broadcast_to(norm, (_RB, D))
    h0 = jnp.dot(x_ref[...], we_ref[...],
                 preferred_element_type=jnp.float32) + be_ref[...]
    hn = h0 * norm
    u0_ref[...] = jnp.dot(hn, wc0_ref[...], preferred_element_type=jnp.float32)
    u1_ref[...] = jnp.dot(hn, wc1_ref[...], preferred_element_type=jnp.float32)


def _tc_prep(x, w_enc, b_enc, deg16, wc0, wc1):
    grid = (N_NODES // _RB,)
    call = pl.pallas_call(
        _prep_body,
        grid=grid,
        in_specs=[
            pl.BlockSpec((_RB, D), lambda i: (i, 0)),
            pl.BlockSpec((D, D), lambda i: (0, 0)),
            pl.BlockSpec((1, D), lambda i: (0, 0)),
            pl.BlockSpec((2, _RB, D), lambda i: (0, i, 0)),
            pl.BlockSpec((D, D), lambda i: (0, 0)),
            pl.BlockSpec((D, D), lambda i: (0, 0)),
        ],
        out_specs=[
            pl.BlockSpec((_RB, D), lambda i: (i, 0)),
            pl.BlockSpec((_RB, D), lambda i: (i, 0)),
            pl.BlockSpec((_RB, D), lambda i: (i, 0)),
        ],
        out_shape=[
            jax.ShapeDtypeStruct((N_NODES, D), jnp.float32),
            jax.ShapeDtypeStruct((N_NODES, D), jnp.float32),
            jax.ShapeDtypeStruct((N_NODES, D), jnp.float32),
        ],
    )
    return call(x, w_enc, b_enc, deg16, wc0, wc1)


def _mid_body(s_ref, a_ref, norm_ref, wep_ref, bc_ref, wc_ref,
              u0_ref, u1_ref):
    nc = norm_ref[...][:, 0:1]
    a16 = a_ref[0] + a_ref[1]
    outs = (u0_ref, u1_ref)
    for i in range(2):
        agg = nc * (s_ref[i] + jnp.dot(a16, wep_ref[i],
                                       preferred_element_type=jnp.float32))
        h = jax.nn.relu(agg + bc_ref[i])
        outs[i][...] = jnp.dot(h * nc, wc_ref[i],
                               preferred_element_type=jnp.float32)


def _tc_mid(s2, a16, norm16, wep, bc, wc):
    grid = (N_NODES // _RB,)
    return pl.pallas_call(
        _mid_body,
        grid=grid,
        in_specs=[
            pl.BlockSpec((2, _RB, D), lambda i: (0, i, 0)),
            pl.BlockSpec((2, _RB, D), lambda i: (0, i, 0)),
            pl.BlockSpec((_RB, D), lambda i: (i, 0)),
            pl.BlockSpec((2, D, D), lambda i: (0, 0, 0)),
            pl.BlockSpec((2, 1, D), lambda i: (0, 0, 0)),
            pl.BlockSpec((2, D, D), lambda i: (0, 0, 0)),
        ],
        out_specs=[
            pl.BlockSpec((_RB, D), lambda i: (i, 0)),
            pl.BlockSpec((_RB, D), lambda i: (i, 0)),
        ],
        out_shape=[
            jax.ShapeDtypeStruct((N_NODES, D), jnp.float32),
            jax.ShapeDtypeStruct((N_NODES, D), jnp.float32),
        ],
    )(s2, a16, norm16, wep, bc, wc)


def _final_body(s_ref, a_ref, norm_ref, wep_ref, bc_ref, batch_ref,
                wpred_ref, bpred_ref, out_ref):
    nc = norm_ref[...][:, 0:1]
    a16 = a_ref[0] + a_ref[1]
    hsum = jnp.zeros((N_NODES, D), jnp.float32)
    for i in range(2):
        agg = nc * (s_ref[i] + jnp.dot(a16, wep_ref[i],
                                       preferred_element_type=jnp.float32))
        hsum = hsum + jax.nn.relu(agg + bc_ref[i])
    gids = lax.broadcasted_iota(jnp.int32, (G_GRAPHS, N_NODES), 0)
    m = (gids == batch_ref[...]).astype(jnp.float32)
    sums = jnp.dot(m, hsum, preferred_element_type=jnp.float32)
    counts = jnp.sum(m, axis=1, keepdims=True)
    readout = sums / jnp.maximum(counts, 1.0)
    out_ref[...] = jnp.dot(readout, wpred_ref[...],
                           preferred_element_type=jnp.float32) + bpred_ref[...]


def _tc_final(s2, a16, norm16, wep, bc, batch2d, wpred_p, bpred_p):
    return pl.pallas_call(
        _final_body,
        out_shape=jax.ShapeDtypeStruct((G_GRAPHS, D), jnp.float32),
    )(s2, a16, norm16, wep, bc, batch2d, wpred_p, bpred_p)


def kernel(x, edge_index, edge_attr, batch,
           W_enc, b_enc, W_conv, b_conv, W_edge, W_pred, b_pred):
    src = edge_index[0]
    dst = edge_index[1]
    pad = E_PAD - N_EDGES
    src_p = jnp.concatenate(
        [src, jnp.zeros((pad,), jnp.int32)]).reshape(NBLK, EB)
    dst_p = jnp.concatenate(
        [dst, jnp.full((pad,), DUMMY_ROW, jnp.int32)]).reshape(NBLK, EB)
    ea_r = jnp.zeros((E_PAD, W16), jnp.float32).at[:N_EDGES, :D_EDGE].set(
        edge_attr).reshape(NBLK * W16, EB)

    deg2 = _deg_kernel(dst_p)[:, :N_NODES]
    norm128, u0, u1 = _tc_prep(x, W_enc, b_enc.reshape(1, D), deg2,
                               W_conv[0, 0], W_conv[1, 0])

    a2 = _edge_agg_kernel(norm128, src_p, dst_p, ea_r)[:, :N_NODES]

    wep = jnp.zeros((2, 2, D, D), jnp.float32).at[:, :, :D_EDGE, :].set(W_edge)
    bc = b_conv.reshape(2, 2, 1, D)

    s2 = _spmm_kernel(u0, u1, src_p, dst_p)[:, :N_NODES]
    u0n, u1n = _tc_mid(s2, a2, norm128, wep[:, 0], bc[:, 0],
                       jnp.stack([W_conv[0, 1], W_conv[1, 1]]))

    s2b = _spmm_kernel(u0n, u1n, src_p, dst_p)[:, :N_NODES]

    wpred_p = jnp.zeros((D, D), jnp.float32).at[:, :NUM_TASKS].set(W_pred)
    bpred_p = jnp.zeros((1, D), jnp.float32).at[0, :NUM_TASKS].set(b_pred)
    out = _tc_final(s2b, a2, norm128, wep[:, 1], bc[:, 1],
                    batch.reshape(1, N_NODES), wpred_p, bpred_p)
    return out[:, :NUM_TASKS]


# speedup vs baseline: 4.6618x; 1.0432x over previous
import functools

import jax
import jax.numpy as jnp
from jax import lax
from jax.experimental import pallas as pl
from jax.experimental.pallas import tpu as pltpu
from jax.experimental.pallas import tpu_sc as plsc

N_NODES = 10000
N_EDGES = 320000
D = 128
D_EDGE = 4
G_GRAPHS = 128
NUM_TASKS = 1

NC = 2
NS = 16
L = 16

EB = 128
NBLK = 2560
E_PAD = NBLK * EB
BPT = NBLK // NS
NPAD = 10112
RPS = NPAD // NS
DUMMY_ROW = N_NODES + 8
W16 = 16

_mesh = plsc.VectorSubcoreMesh(core_axis_name="c", subcore_axis_name="s")


def _zero_rows(ref, nrows, width):
    @pl.loop(0, nrows)
    def _(r):
        for j in range(width // L):
            ref[r, pl.ds(j * L, L)] = jnp.zeros((L,), jnp.float32)


def _zero_acc_slice(zsrc, acc, base):
    for k in range(RPS // EB):
        pltpu.sync_copy(zsrc, acc.at[pl.ds(base + k * EB, EB)])
    rem = RPS % EB
    if rem:
        pltpu.sync_copy(zsrc.at[pl.ds(0, rem)],
                        acc.at[pl.ds(base + (RPS // EB) * EB, rem)])


CHUNK = 16
BPT2 = NBLK // (NC * NS)


@functools.partial(
    pl.kernel,
    out_type=jax.ShapeDtypeStruct((NC, NPAD, D), jnp.float32),
    mesh=_mesh,
    scratch_types=[
        pltpu.VMEM((CHUNK, EB), jnp.int32),
        pltpu.VMEM((EB, D), jnp.float32),
        pltpu.VMEM_SHARED((NPAD, D), jnp.float32),
    ],
)
def _deg_kernel(dst_hbm, deg_hbm, dstv, ones, acc):
    c = lax.axis_index("c")
    s = lax.axis_index("s")
    w = s * NC + c
    _zero_rows(ones, EB, D)
    _zero_acc_slice(ones, acc, s * RPS)

    @pl.loop(0, EB)
    def _(r):
        for j in range(D // L):
            ones[r, pl.ds(j * L, L)] = jnp.full((L,), 1.0, jnp.float32)

    plsc.subcore_barrier()

    @pl.loop(0, BPT2 // CHUNK)
    def _(g):
        pltpu.sync_copy(dst_hbm.at[pl.ds(w * BPT2 + g * CHUNK, CHUNK)], dstv)

        @pl.loop(0, CHUNK)
        def _(b):
            pltpu.sync_copy(ones, acc.at[dstv.at[b]], add=True)

    plsc.subcore_barrier()

    @pl.when(c == 0)
    def _():
        pltpu.sync_copy(acc.at[pl.ds(s * RPS, RPS)],
                        deg_hbm.at[0, pl.ds(s * RPS, RPS)])

    @pl.when(c == 1)
    def _():
        pltpu.sync_copy(acc.at[pl.ds(s * RPS, RPS)],
                        deg_hbm.at[1, pl.ds(s * RPS, RPS)])


@functools.partial(
    pl.kernel,
    out_type=jax.ShapeDtypeStruct((NC, NPAD, D), jnp.float32),
    mesh=_mesh,
    scratch_types=[
        pltpu.VMEM((CHUNK, EB), jnp.int32),
        pltpu.VMEM((CHUNK, EB), jnp.int32),
        pltpu.VMEM((EB, D), jnp.float32),
        pltpu.VMEM((W16, EB), jnp.float32),
        pltpu.VMEM((EB, D), jnp.float32),
        pltpu.SemaphoreType.DMA,
        pltpu.SemaphoreType.DMA,
        pltpu.VMEM_SHARED((NPAD, D), jnp.float32),
    ],
)
def _edge_agg_kernel(norm_hbm, src_hbm, dst_hbm, ea_hbm, a_hbm,
                     srcv, dstv, nr, eav, t2, gsem, ssem, acc):
    c = lax.axis_index("c")
    s = lax.axis_index("s")
    w = s * NC + c
    _zero_rows(t2, EB, D)
    _zero_acc_slice(t2, acc, s * RPS)
    plsc.subcore_barrier()

    @pl.loop(0, BPT2 // CHUNK)
    def _(g):
        base = w * BPT2 + g * CHUNK
        pltpu.sync_copy(src_hbm.at[pl.ds(base, CHUNK)], srcv)
        pltpu.sync_copy(dst_hbm.at[pl.ds(base, CHUNK)], dstv)
        pltpu.async_copy(norm_hbm.at[srcv.at[0]], nr, gsem)

        @pl.loop(0, CHUNK)
        def _(b):
            pltpu.sync_copy(ea_hbm.at[pl.ds((base + b) * W16, W16)], eav)
            pltpu.make_async_copy(norm_hbm.at[srcv.at[b]], nr, gsem).wait()

            @pl.when(b > 0)
            def _():
                pltpu.make_async_copy(t2, acc.at[dstv.at[b]], ssem).wait()

            @pl.loop(0, W16)
            def _(r):
                for q in range(EB // W16):
                    e = r * (EB // W16) + q
                    t2[e, pl.ds(0, W16)] = (
                        nr[e, pl.ds(0, W16)] * eav[r, pl.ds(q * W16, W16)])

            pltpu.async_copy(t2, acc.at[dstv.at[b]], ssem, add=True)

            @pl.when(b + 1 < CHUNK)
            def _():
                pltpu.async_copy(norm_hbm.at[srcv.at[b + 1]], nr, gsem)

        pltpu.make_async_copy(t2, acc.at[dstv.at[0]], ssem).wait()

    plsc.subcore_barrier()

    @pl.when(c == 0)
    def _():
        pltpu.sync_copy(acc.at[pl.ds(s * RPS, RPS)],
                        a_hbm.at[0, pl.ds(s * RPS, RPS)])

    @pl.when(c == 1)
    def _():
        pltpu.sync_copy(acc.at[pl.ds(s * RPS, RPS)],
                        a_hbm.at[1, pl.ds(s * RPS, RPS)])


@functools.partial(
    pl.kernel,
    out_type=jax.ShapeDtypeStruct((NC, NPAD, D), jnp.float32),
    mesh=_mesh,
    scratch_types=[
        pltpu.VMEM((CHUNK, EB), jnp.int32),
        pltpu.VMEM((CHUNK, EB), jnp.int32),
        pltpu.VMEM((EB, D), jnp.float32),
        pltpu.VMEM((EB, D), jnp.float32),
        pltpu.SemaphoreType.DMA,
        pltpu.SemaphoreType.DMA,
        pltpu.SemaphoreType.DMA,
        pltpu.SemaphoreType.DMA,
        pltpu.VMEM_SHARED((NPAD, D), jnp.float32),
    ],
)
def _spmm_kernel(u0_hbm, u1_hbm, src_hbm, dst_hbm, s_hbm,
                 srcv, dstv, rows0, rows1, gsem0, gsem1, ssem0, ssem1, acc):
    c = lax.axis_index("c")
    s = lax.axis_index("s")
    _zero_rows(rows0, EB, D)
    _zero_acc_slice(rows0, acc, s * RPS)
    plsc.subcore_barrier()

    def run(u_hbm):
        @pl.loop(0, BPT // CHUNK)
        def _(g):
            base = s * BPT + g * CHUNK
            pltpu.sync_copy(src_hbm.at[pl.ds(base, CHUNK)], srcv)
            pltpu.sync_copy(dst_hbm.at[pl.ds(base, CHUNK)], dstv)
            pltpu.async_copy(u_hbm.at[srcv.at[0]], rows0, gsem0)

            @pl.loop(0, CHUNK, step=2)
            def _(b):
                @pl.when(b > 0)
                def _():
                    pltpu.make_async_copy(
                        rows1, acc.at[dstv.at[b]], ssem1).wait()
                pltpu.async_copy(u_hbm.at[srcv.at[b + 1]], rows1, gsem1)
                pltpu.make_async_copy(
                    u_hbm.at[srcv.at[b]], rows0, gsem0).wait()
                pltpu.async_copy(rows0, acc.at[dstv.at[b]], ssem0, add=True)

                @pl.when(b + 2 < CHUNK)
                def _():
                    pltpu.make_async_copy(
                        rows0, acc.at[dstv.at[b]], ssem0).wait()
                    pltpu.async_copy(u_hbm.at[srcv.at[b + 2]], rows0, gsem0)
                pltpu.make_async_copy(
                    u_hbm.at[srcv.at[b + 1]], rows1, gsem1).wait()
                pltpu.async_copy(rows1, acc.at[dstv.at[b + 1]], ssem1, add=True)

            pltpu.make_async_copy(rows0, acc.at[dstv.at[0]], ssem0).wait()
            pltpu.make_async_copy(rows1, acc.at[dstv.at[1]], ssem1).wait()

    @pl.when(c == 0)
    def _():
        run(u0_hbm)

    @pl.when(c == 1)
    def _():
        run(u1_hbm)

    plsc.subcore_barrier()

    @pl.when(c == 0)
    def _():
        pltpu.sync_copy(acc.at[pl.ds(s * RPS, RPS)],
                        s_hbm.at[0, pl.ds(s * RPS, RPS)])

    @pl.when(c == 1)
    def _():
        pltpu.sync_copy(acc.at[pl.ds(s * RPS, RPS)],
                        s_hbm.at[1, pl.ds(s * RPS, RPS)])


_RB = 1000


def _prep_body(x_ref, we_ref, be_ref, deg_ref, wc0_ref, wc1_ref,
               norm_ref, u0_ref, u1_ref):
    deg = deg_ref[0][:, 0:1] + deg_ref[1][:, 0:1]
    norm = lax.rsqrt(jnp.maximum(deg, 1.0))
    norm_ref[...] = jnp.broadcast_to(norm, (_RB, D))
    h0 = jnp.dot(x_ref[...], we_ref[...],
                 preferred_element_type=jnp.float32) + be_ref[...]
    hn = h0 * norm
    u0_ref[...] = jnp.dot(hn, wc0_ref[...], preferred_element_type=jnp.float32)
    u1_ref[...] = jnp.dot(hn, wc1_ref[...], preferred_element_type=jnp.float32)


def _tc_prep(x, w_enc, b_enc, deg16, wc0, wc1):
    grid = (N_NODES // _RB,)
    call = pl.pallas_call(
        _prep_body,
        grid=grid,
        in_specs=[
            pl.BlockSpec((_RB, D), lambda i: (i, 0)),
            pl.BlockSpec((D, D), lambda i: (0, 0)),
            pl.BlockSpec((1, D), lambda i: (0, 0)),
            pl.BlockSpec((2, _RB, D), lambda i: (0, i, 0)),
            pl.BlockSpec((D, D), lambda i: (0, 0)),
            pl.BlockSpec((D, D), lambda i: (0, 0)),
        ],
        out_specs=[
            pl.BlockSpec((_RB, D), lambda i: (i, 0)),
            pl.BlockSpec((_RB, D), lambda i: (i, 0)),
            pl.BlockSpec((_RB, D), lambda i: (i, 0)),
        ],
        out_shape=[
            jax.ShapeDtypeStruct((N_NODES, D), jnp.float32),
            jax.ShapeDtypeStruct((N_NODES, D), jnp.float32),
            jax.ShapeDtypeStruct((N_NODES, D), jnp.float32),
        ],
    )
    return call(x, w_enc, b_enc, deg16, wc0, wc1)


def _mid_body(s_ref, a_ref, norm_ref, wep_ref, bc_ref, wc_ref,
              u0_ref, u1_ref):
    nc = norm_ref[...][:, 0:1]
    a16 = a_ref[0] + a_ref[1]
    outs = (u0_ref, u1_ref)
    for i in range(2):
        agg = nc * (s_ref[i] + jnp.dot(a16, wep_ref[i],
                                       preferred_element_type=jnp.float32))
        h = jax.nn.relu(agg + bc_ref[i])
        outs[i][...] = jnp.dot(h * nc, wc_ref[i],
                               preferred_element_type=jnp.float32)


def _tc_mid(s2, a16, norm16, wep, bc, wc):
    grid = (N_NODES // _RB,)
    return pl.pallas_call(
        _mid_body,
        grid=grid,
        in_specs=[
            pl.BlockSpec((2, _RB, D), lambda i: (0, i, 0)),
            pl.BlockSpec((2, _RB, D), lambda i: (0, i, 0)),
            pl.BlockSpec((_RB, D), lambda i: (i, 0)),
            pl.BlockSpec((2, D, D), lambda i: (0, 0, 0)),
            pl.BlockSpec((2, 1, D), lambda i: (0, 0, 0)),
            pl.BlockSpec((2, D, D), lambda i: (0, 0, 0)),
        ],
        out_specs=[
            pl.BlockSpec((_RB, D), lambda i: (i, 0)),
            pl.BlockSpec((_RB, D), lambda i: (i, 0)),
        ],
        out_shape=[
            jax.ShapeDtypeStruct((N_NODES, D), jnp.float32),
            jax.ShapeDtypeStruct((N_NODES, D), jnp.float32),
        ],
    )(s2, a16, norm16, wep, bc, wc)


def _final_body(s_ref, a_ref, norm_ref, wep_ref, bc_ref, batch_ref,
                wpred_ref, bpred_ref, out_ref):
    nc = norm_ref[...][:, 0:1]
    a16 = a_ref[0] + a_ref[1]
    hsum = jnp.zeros((N_NODES, D), jnp.float32)
    for i in range(2):
        agg = nc * (s_ref[i] + jnp.dot(a16, wep_ref[i],
                                       preferred_element_type=jnp.float32))
        hsum = hsum + jax.nn.relu(agg + bc_ref[i])
    gids = lax.broadcasted_iota(jnp.int32, (G_GRAPHS, N_NODES), 0)
    m = (gids == batch_ref[...]).astype(jnp.float32)
    sums = jnp.dot(m, hsum, preferred_element_type=jnp.float32)
    counts = jnp.sum(m, axis=1, keepdims=True)
    readout = sums / jnp.maximum(counts, 1.0)
    out_ref[...] = jnp.dot(readout, wpred_ref[...],
                           preferred_element_type=jnp.float32) + bpred_ref[...]


def _tc_final(s2, a16, norm16, wep, bc, batch2d, wpred_p, bpred_p):
    return pl.pallas_call(
        _final_body,
        out_shape=jax.ShapeDtypeStruct((G_GRAPHS, D), jnp.float32),
    )(s2, a16, norm16, wep, bc, batch2d, wpred_p, bpred_p)


def kernel(x, edge_index, edge_attr, batch,
           W_enc, b_enc, W_conv, b_conv, W_edge, W_pred, b_pred):
    src = edge_index[0]
    dst = edge_index[1]
    pad = E_PAD - N_EDGES
    src_p = jnp.concatenate(
        [src, jnp.zeros((pad,), jnp.int32)]).reshape(NBLK, EB)
    dst_p = jnp.concatenate(
        [dst, jnp.full((pad,), DUMMY_ROW, jnp.int32)]).reshape(NBLK, EB)
    ea_r = jnp.zeros((E_PAD, W16), jnp.float32).at[:N_EDGES, :D_EDGE].set(
        edge_attr).reshape(NBLK * W16, EB)

    deg2 = _deg_kernel(dst_p)[:, :N_NODES]
    norm128, u0, u1 = _tc_prep(x, W_enc, b_enc.reshape(1, D), deg2,
                               W_conv[0, 0], W_conv[1, 0])

    a2 = _edge_agg_kernel(norm128, src_p, dst_p, ea_r)[:, :N_NODES]

    wep = jnp.zeros((2, 2, D, D), jnp.float32).at[:, :, :D_EDGE, :].set(W_edge)
    bc = b_conv.reshape(2, 2, 1, D)

    s2 = _spmm_kernel(u0, u1, src_p, dst_p)[:, :N_NODES]
    u0n, u1n = _tc_mid(s2, a2, norm128, wep[:, 0], bc[:, 0],
                       jnp.stack([W_conv[0, 1], W_conv[1, 1]]))

    s2b = _spmm_kernel(u0n, u1n, src_p, dst_p)[:, :N_NODES]

    wpred_p = jnp.zeros((D, D), jnp.float32).at[:, :NUM_TASKS].set(W_pred)
    bpred_p = jnp.zeros((1, D), jnp.float32).at[0, :NUM_TASKS].set(b_pred)
    out = _tc_final(s2b, a2, norm128, wep[:, 1], bc[:, 1],
                    batch.reshape(1, N_NODES), wpred_p, bpred_p)
    return out[:, :NUM_TASKS]
